# Initial kernel scaffold; baseline (speedup 1.0000x reference)
import jax
import jax.numpy as jnp
from jax.experimental import pallas as pl


def kernel(logits, temperatures, top_ps, top_ks):
    raise NotImplementedError("write your pallas kernel here")


# speedup vs baseline: 45.1866x; 45.1866x over previous
import jax
import jax.numpy as jnp
from jax.experimental import pallas as pl

_NEG = -1e30
_TEMP_EPS = 1e-5
_ROWS = 8


def _diag_col(ref, dtype):
    v = ref[0]
    r = v.shape[-1]
    bc = jnp.broadcast_to(v, (r, r))
    i0 = jax.lax.broadcasted_iota(jnp.int32, (r, r), 0)
    i1 = jax.lax.broadcasted_iota(jnp.int32, (r, r), 1)
    z = jnp.zeros((), dtype)
    return jnp.sum(jnp.where(i0 == i1, bc, z), axis=1, keepdims=True)


def _sortable(f):
    u = jax.lax.bitcast_convert_type(f, jnp.int32)
    return jnp.where(u < 0, u ^ jnp.int32(0x7FFFFFFF), u)


def _sampler_body(lg_ref, t_ref, p_ref, k_ref, probs_ref, lp_ref, tok_ref,
                  cl_ref):
    R, V = lg_ref.shape
    t = _diag_col(t_ref, jnp.float32)
    top_p = _diag_col(p_ref, jnp.float32)
    k = jnp.clip(_diag_col(k_ref, jnp.int32), 1, V)

    t = jnp.where(t < _TEMP_EPS, 1.0, t)
    x = lg_ref[...] / t
    m = jnp.max(x, axis=-1, keepdims=True)
    e = jnp.exp(x - m)
    z = jnp.sum(e, axis=-1, keepdims=True)
    s = _sortable(x)

    lo0 = _sortable(jnp.min(x, axis=-1, keepdims=True))
    hi0 = _sortable(m)
    pz = top_p * z

    def step(_, carry):
        lo_a, hi_a, lo_b, hi_b = carry
        mid_a = (lo_a >> 1) + (hi_a >> 1) + ((lo_a | hi_a) & 1)
        cnt = jnp.sum((s >= mid_a).astype(jnp.int32), axis=-1, keepdims=True)
        ok_a = cnt >= k
        lo_a = jnp.where(ok_a, mid_a, lo_a)
        hi_a = jnp.where(ok_a, hi_a, mid_a - 1)
        mid_b = (lo_b >> 1) + (hi_b >> 1) + ((lo_b & hi_b) & 1)
        g = jnp.sum(jnp.where(s > mid_b, e, jnp.float32(0.0)), axis=-1,
                    keepdims=True)
        ok_b = g <= pz
        hi_b = jnp.where(ok_b, mid_b, hi_b)
        lo_b = jnp.where(ok_b, lo_b, mid_b + 1)
        return lo_a, hi_a, lo_b, hi_b

    lo_a, _, lo_b, _ = jax.lax.fori_loop(0, 32, step, (lo0, hi0, lo0, hi0))
    cutoff = jnp.maximum(lo_a, lo_b)
    kept = s >= cutoff

    z2 = jnp.sum(jnp.where(kept, e, jnp.float32(0.0)), axis=-1, keepdims=True)
    lse = m + jnp.log(z2)
    lp = jnp.where(kept, x - lse, _NEG - lse)
    probs = jnp.exp(lp)
    lp_ref[...] = lp
    probs_ref[...] = probs

    iota = jax.lax.broadcasted_iota(jnp.int32, (R, V), 1)
    pm = jnp.max(probs, axis=-1, keepdims=True)
    tok_ref[...] = jnp.min(jnp.where(probs == pm, iota, V), axis=-1,
                           keepdims=True)
    cl_ref[...] = m - lse


def _run(logits, t3, p3, k3):
    B, V = logits.shape
    G = B // _ROWS
    out_shape = [
        jax.ShapeDtypeStruct((B, V), jnp.float32),
        jax.ShapeDtypeStruct((B, V), jnp.float32),
        jax.ShapeDtypeStruct((B, 1), jnp.int32),
        jax.ShapeDtypeStruct((B, 1), jnp.float32),
    ]
    return pl.pallas_call(
        _sampler_body,
        grid=(G,),
        in_specs=[
            pl.BlockSpec((_ROWS, V), lambda i: (i, 0)),
            pl.BlockSpec((1, 1, _ROWS), lambda i: (i, 0, 0)),
            pl.BlockSpec((1, 1, _ROWS), lambda i: (i, 0, 0)),
            pl.BlockSpec((1, 1, _ROWS), lambda i: (i, 0, 0)),
        ],
        out_specs=[
            pl.BlockSpec((_ROWS, V), lambda i: (i, 0)),
            pl.BlockSpec((_ROWS, V), lambda i: (i, 0)),
            pl.BlockSpec((_ROWS, 1), lambda i: (i, 0)),
            pl.BlockSpec((_ROWS, 1), lambda i: (i, 0)),
        ],
        out_shape=out_shape,
    )(logits, t3, p3, k3)


def kernel(logits, temperatures, top_ps, top_ks):
    B, V = logits.shape
    G = B // _ROWS
    t3 = temperatures.reshape(G, 1, _ROWS)
    p3 = top_ps.reshape(G, 1, _ROWS)
    k3 = top_ks.astype(jnp.int32).reshape(G, 1, _ROWS)
    probs, lp, tok, cl = _run(logits, t3, p3, k3)
    return probs, lp, tok.reshape(B), cl.reshape(B)
